# baseline (speedup 1.0000x reference)
import functools

import jax
import jax.numpy as jnp
from jax import lax
from jax.experimental import pallas as pl
from jax.experimental.pallas import tpu as pltpu
from jax.experimental.pallas import tpu_sc as plsc

B = 16384
CTX = 128
G = 64
D = 3000
DP = 3008

BB = 1024
NB = B // BB

NC = 2
NS = 16
NW = NC * NS
BPW = B // NW
CH = 16
NCHUNK = BPW // CH


def _argmin_body(x_ref, c_ref, out_ref):
    xb = x_ref[...]
    best = jnp.full((BB,), jnp.inf, dtype=jnp.float32)
    bidx = jnp.zeros((BB,), dtype=jnp.int32)
    for g in range(G):
        diff = c_ref[g, :][None, :] - xb
        dist = jnp.sqrt(jnp.sum(diff * diff, axis=1))
        upd = dist < best
        best = jnp.where(upd, dist, best)
        bidx = jnp.where(upd, g, bidx)
    out_ref[...] = bidx.reshape(1, 1, BB)


def _nearest(x, centers):
    out3 = pl.pallas_call(
        _argmin_body,
        grid=(NB,),
        in_specs=[
            pl.BlockSpec((BB, CTX), lambda i: (i, 0)),
            pl.BlockSpec((G, CTX), lambda i: (0, 0)),
        ],
        out_specs=pl.BlockSpec((1, 1, BB), lambda i: (i, 0, 0)),
        out_shape=jax.ShapeDtypeStruct((NB, 1, BB), jnp.int32),
    )(x, centers)
    return out3.reshape(B)


@functools.cache
def _make_gather():
    mesh = plsc.VectorSubcoreMesh(core_axis_name="c", subcore_axis_name="s")

    @functools.partial(
        pl.kernel,
        mesh=mesh,
        out_type=jax.ShapeDtypeStruct((B, D), jnp.float32),
        scratch_types=[
            pltpu.VMEM((CH,), jnp.int32),
            pltpu.VMEM((CH, D), jnp.float32),
            pltpu.SemaphoreType.DMA,
        ],
        compiler_params=pltpu.CompilerParams(use_tc_tiling_on_sc=False),
    )
    def _gather(table_hbm, idx_hbm, out_hbm, idx_c, rows_v, sem):
        wid = lax.axis_index("s") * NC + lax.axis_index("c")
        base = wid * BPW

        def chunk(j, carry):
            pltpu.sync_copy(idx_hbm.at[wid, j], idx_c)
            pltpu.async_copy(table_hbm.at[idx_c], rows_v, sem).wait()
            pltpu.sync_copy(rows_v, out_hbm.at[pl.ds(base + j * CH, CH)])
            return carry

        lax.fori_loop(0, NCHUNK, chunk, 0)

    return _gather


def kernel(x, centers, center_values):
    nearest = _nearest(x, centers)
    table = jnp.transpose(center_values)
    idx3 = nearest.reshape(NW, NCHUNK, CH)
    return _make_gather()(table, idx3)


# speedup vs baseline: 1.0155x; 1.0155x over previous
import functools

import jax
import jax.numpy as jnp
from jax import lax
from jax.experimental import pallas as pl
from jax.experimental.pallas import tpu as pltpu
from jax.experimental.pallas import tpu_sc as plsc

B = 16384
CTX = 128
G = 64
D = 3000
DP = 3008

BB = 1024
NB = B // BB

NC = 2
NS = 16
NW = NC * NS
BPW = B // NW
CH = 16
NCHUNK = BPW // CH


def _argmin_body(x_ref, c_ref, out_ref):
    xb = x_ref[...]
    best = jnp.full((BB,), jnp.inf, dtype=jnp.float32)
    bidx = jnp.zeros((BB,), dtype=jnp.int32)
    for g in range(G):
        diff = c_ref[g, :][None, :] - xb
        dist = jnp.sqrt(jnp.sum(diff * diff, axis=1))
        upd = dist < best
        best = jnp.where(upd, dist, best)
        bidx = jnp.where(upd, g, bidx)
    out_ref[...] = bidx.reshape(1, 1, BB)


def _nearest(x, centers):
    out3 = pl.pallas_call(
        _argmin_body,
        grid=(NB,),
        in_specs=[
            pl.BlockSpec((BB, CTX), lambda i: (i, 0)),
            pl.BlockSpec((G, CTX), lambda i: (0, 0)),
        ],
        out_specs=pl.BlockSpec((1, 1, BB), lambda i: (i, 0, 0)),
        out_shape=jax.ShapeDtypeStruct((NB, 1, BB), jnp.int32),
    )(x, centers)
    return out3.reshape(B)


def _transpose_body(cv_ref, out_ref):
    out_ref[...] = jnp.transpose(cv_ref[...])


def _transpose(center_values):
    return pl.pallas_call(
        _transpose_body,
        in_specs=[pl.BlockSpec((3 * 1000, G), lambda: (0, 0))],
        out_specs=pl.BlockSpec((G, D), lambda: (0, 0)),
        out_shape=jax.ShapeDtypeStruct((G, D), jnp.float32),
    )(center_values)


@functools.cache
def _make_gather():
    mesh = plsc.VectorSubcoreMesh(core_axis_name="c", subcore_axis_name="s")

    @functools.partial(
        pl.kernel,
        mesh=mesh,
        out_type=jax.ShapeDtypeStruct((B, D), jnp.float32),
        scratch_types=[
            pltpu.VMEM((CH,), jnp.int32),
            pltpu.VMEM((CH, D), jnp.float32),
            pltpu.SemaphoreType.DMA,
        ],
        compiler_params=pltpu.CompilerParams(use_tc_tiling_on_sc=False),
    )
    def _gather(table_hbm, idx_hbm, out_hbm, idx_c, rows_v, sem):
        wid = lax.axis_index("s") * NC + lax.axis_index("c")
        base = wid * BPW

        def chunk(j, carry):
            pltpu.sync_copy(idx_hbm.at[wid, j], idx_c)
            pltpu.async_copy(table_hbm.at[idx_c], rows_v, sem).wait()
            pltpu.sync_copy(rows_v, out_hbm.at[pl.ds(base + j * CH, CH)])
            return carry

        lax.fori_loop(0, NCHUNK, chunk, 0)

    return _gather


def kernel(x, centers, center_values):
    nearest = _nearest(x, centers)
    table = _transpose(center_values)
    idx3 = nearest.reshape(NW, NCHUNK, CH)
    return _make_gather()(table, idx3)


# speedup vs baseline: 1.3622x; 1.3414x over previous
import functools

import jax
import jax.numpy as jnp
from jax import lax
from jax.experimental import pallas as pl
from jax.experimental.pallas import tpu as pltpu
from jax.experimental.pallas import tpu_sc as plsc

B = 16384
CTX = 128
G = 64
D = 3000

BB = 1024
NB = B // BB

NC = 2
NS = 16
NWB = 8
NWK = 4
BW = B // NWB
JW = BW // 128
KW = 752
IT = KW // 8
ROW = JW * 8 * 128
ZN = (D // 8) * 128 * 8 * 128


def _argmin_body(x_ref, c_ref, out_ref):
    xb = x_ref[...]
    best = jnp.full((BB,), jnp.inf, dtype=jnp.float32)
    bidx = jnp.zeros((BB,), dtype=jnp.int32)
    for g in range(G):
        diff = c_ref[g, :][None, :] - xb
        dist = jnp.sqrt(jnp.sum(diff * diff, axis=1))
        upd = dist < best
        best = jnp.where(upd, dist, best)
        bidx = jnp.where(upd, g, bidx)
    out_ref[...] = bidx.reshape(1, 1, BB)


def _nearest(x, centers):
    out3 = pl.pallas_call(
        _argmin_body,
        grid=(NB,),
        in_specs=[
            pl.BlockSpec((BB, CTX), lambda i: (i, 0)),
            pl.BlockSpec((G, CTX), lambda i: (0, 0)),
        ],
        out_specs=pl.BlockSpec((1, 1, BB), lambda i: (i, 0, 0)),
        out_shape=jax.ShapeDtypeStruct((NB, 1, BB), jnp.int32),
    )(x, centers)
    return out3.reshape(B)


@functools.cache
def _make_gather():
    mesh = plsc.VectorSubcoreMesh(core_axis_name="c", subcore_axis_name="s")

    @functools.partial(
        pl.kernel,
        mesh=mesh,
        out_type=jax.ShapeDtypeStruct((ZN,), jnp.float32),
        scratch_types=[
            pltpu.VMEM((KW, G), jnp.float32),
            pltpu.VMEM((BW,), jnp.int32),
            pltpu.VMEM((ROW,), jnp.float32),
            pltpu.VMEM((ROW,), jnp.float32),
            pltpu.SemaphoreType.DMA,
            pltpu.SemaphoreType.DMA,
        ],
        compiler_params=pltpu.CompilerParams(
            use_tc_tiling_on_sc=False, needs_layout_passes=False),
    )
    def _gather(cv_hbm, idx_hbm, z_hbm, tab_v, idx_v, buf0, buf1, sem0, sem1):
        wid = lax.axis_index("s") * NC + lax.axis_index("c")
        wb = wid % NWB
        wk = wid // NWB
        b0 = wb * BW
        j0 = wb * JW
        k0 = jnp.minimum(wk * KW, D - KW)
        i0 = k0 // 8
        pltpu.sync_copy(cv_hbm.at[pl.ds(k0, KW)], tab_v)
        pltpu.sync_copy(idx_hbm.at[pl.ds(b0, BW)], idx_v)

        def zoff(i):
            return ((i0 + i) * 128 + j0) * 1024

        def fill(i, buf):
            kvs = [jnp.broadcast_to(i * 8 + r, (16,)).astype(jnp.int32)
                   for r in range(8)]

            def jbody(jj, carry):
                base = jj * 1024
                gvs = [idx_v[pl.ds(jj * 128 + 16 * c, 16)] for c in range(8)]
                for r in range(8):
                    for c in range(8):
                        vals = plsc.load_gather(tab_v, [kvs[r], gvs[c]])
                        buf[pl.ds(base + r * 128 + 16 * c, 16)] = vals
                return carry

            lax.fori_loop(0, JW, jbody, 0)

        fill(0, buf0)
        pltpu.async_copy(buf0, z_hbm.at[pl.ds(zoff(0), ROW)], sem0)
        fill(1, buf1)
        pltpu.async_copy(buf1, z_hbm.at[pl.ds(zoff(1), ROW)], sem1)

        def body(t, carry):
            i = 2 * t
            pltpu.make_async_copy(
                buf0, z_hbm.at[pl.ds(zoff(i - 2), ROW)], sem0).wait()
            fill(i, buf0)
            pltpu.async_copy(buf0, z_hbm.at[pl.ds(zoff(i), ROW)], sem0)
            pltpu.make_async_copy(
                buf1, z_hbm.at[pl.ds(zoff(i - 1), ROW)], sem1).wait()
            fill(i + 1, buf1)
            pltpu.async_copy(buf1, z_hbm.at[pl.ds(zoff(i + 1), ROW)], sem1)
            return carry

        lax.fori_loop(1, IT // 2, body, 0)
        pltpu.make_async_copy(
            buf0, z_hbm.at[pl.ds(zoff(IT - 2), ROW)], sem0).wait()
        pltpu.make_async_copy(
            buf1, z_hbm.at[pl.ds(zoff(IT - 1), ROW)], sem1).wait()

    return _gather


def kernel(x, centers, center_values):
    nearest = _nearest(x, centers)
    z = _make_gather()(center_values, nearest)
    z4 = z.reshape(D // 8, 128, 8, 128)
    return jnp.transpose(z4, (1, 3, 0, 2)).reshape(B, D)
